# Initial kernel scaffold; baseline (speedup 1.0000x reference)
import jax
import jax.numpy as jnp
from jax.experimental import pallas as pl


def kernel(q_prime, spatial_n, spatial_q, length, slope, top_width, side_slope, x_storage, edge_index):
    raise NotImplementedError("write your pallas kernel here")


# speedup vs baseline: 63.4294x; 63.4294x over previous
import functools

import jax
import jax.numpy as jnp
from jax import lax
from jax.experimental import pallas as pl
from jax.experimental.pallas import tpu as pltpu
from jax.experimental.pallas import tpu_sc as plsc

N = 10000
NPAD = 10240
E = 160000
T = 9
LEVELS = 32
BLOCK = (N + LEVELS - 1) // LEVELS
DT = 3600.0
LB = 1e-4
CHUNK = 2000
EPAD = E + 2048
NG = NPAD // 16
LN2 = 0.6931471805599453
SQRT2 = 1.4142135623730951
I32 = jnp.int32
F32 = jnp.float32


def _vln(x):
    bits = lax.bitcast_convert_type(x, I32)
    e = lax.shift_right_arithmetic(bits, 23) - 127
    m = lax.bitcast_convert_type((bits & 0x7FFFFF) | 0x3F800000, F32)
    big = m > SQRT2
    m = jnp.where(big, m * 0.5, m)
    e = jnp.where(big, e + 1, e)
    z = (m - 1.0) / (m + 1.0)
    z2 = z * z
    p = 1.0 + z2 * (1.0 / 3.0 + z2 * (1.0 / 5.0 + z2 * (1.0 / 7.0 + z2 * (1.0 / 9.0))))
    return e.astype(F32) * LN2 + 2.0 * z * p


def _vsqrt(x):
    return jnp.exp(0.5 * _vln(x))


def _sget(ref, i):
    return jnp.max(plsc.load_gather(ref, [jnp.full((16,), i, I32)]))


_INTERPRET = False
_sc_mesh = plsc.VectorSubcoreMesh(core_axis_name="c", subcore_axis_name="s",
                                  num_cores=2, num_subcores=16)


@functools.partial(
    pl.kernel,
    out_type=jax.ShapeDtypeStruct((16,), F32),
    mesh=_sc_mesh,
    scratch_types=[
        pltpu.VMEM((NPAD,), F32),
        pltpu.VMEM((NPAD,), F32),
        pltpu.VMEM((NPAD,), F32),
        pltpu.VMEM((NPAD,), F32),
        pltpu.VMEM((NPAD,), F32),
        pltpu.VMEM((NPAD,), F32),
        pltpu.VMEM((NPAD,), F32),
        pltpu.VMEM((NPAD,), F32),
        pltpu.VMEM((NPAD,), F32),
        pltpu.VMEM((NPAD,), F32),
        pltpu.VMEM((NPAD,), F32),
        pltpu.VMEM((NPAD,), F32),
        pltpu.VMEM((CHUNK,), I32),
        pltpu.VMEM((CHUNK,), I32),
        pltpu.VMEM((48,), I32),
        pltpu.VMEM((16,), F32),
    ],
    compiler_params=pltpu.CompilerParams(needs_layout_passes=False),
    interpret=_INTERPRET,
)
def _route_sc(qp_hbm, n_hbm, q_hbm, len_hbm, s_hbm, tw_hbm, ss_hbm, xs_hbm,
              src_hbm, dst_hbm, off_hbm, out_hbm,
              x_v, acc_v, c1n_v, ql_v, A1_v, e1_v, iv_v, tw_v, ss2_v, ssb_v,
              L1_v, L2_v, se_v, de_v, off_v, outs_v):
    cid = lax.axis_index("c")
    sid = lax.axis_index("s")

    @pl.when(jnp.logical_and(cid == 0, sid == 0))
    def _():
        iota = lax.iota(I32, 16)
        zf16 = jnp.zeros((16,), F32)
        pltpu.sync_copy(off_hbm, off_v)

        pltpu.sync_copy(n_hbm, acc_v)
        pltpu.sync_copy(q_hbm, c1n_v)
        pltpu.sync_copy(s_hbm, ql_v)

        def pre1(i, _):
            gi = i * 16 + iota
            n_man = plsc.load_gather(acc_v, [gi]) * 0.29 + 0.01
            qsp = plsc.load_gather(c1n_v, [gi]) * 3.0
            s0 = jnp.maximum(plsc.load_gather(ql_v, [gi]), 1e-4)
            sq = _vsqrt(s0)
            plsc.store_scatter(A1_v, [gi], n_man * (qsp + 1.0) / (21.0 * sq + 1e-8))
            plsc.store_scatter(e1_v, [gi], 3.0 / (5.0 + 3.0 * qsp))
            plsc.store_scatter(iv_v, [gi], sq / n_man)
            return 0

        lax.fori_loop(0, NG, pre1, 0)
        pltpu.sync_copy(tw_hbm, tw_v)
        pltpu.sync_copy(ss_hbm, acc_v)

        def pre2(i, _):
            gi = i * 16 + iota
            ss = plsc.load_gather(acc_v, [gi])
            plsc.store_scatter(ss2_v, [gi], 2.0 * ss)
            plsc.store_scatter(ssb_v, [gi], 2.0 * _vsqrt(1.0 + ss * ss))
            return 0

        lax.fori_loop(0, NG, pre2, 0)
        pltpu.sync_copy(len_hbm, acc_v)
        pltpu.sync_copy(xs_hbm, c1n_v)

        def pre3(i, _):
            gi = i * 16 + iota
            ln = plsc.load_gather(acc_v, [gi])
            xs = plsc.load_gather(c1n_v, [gi])
            plsc.store_scatter(L1_v, [gi], 2.0 * ln * (1.0 - xs))
            plsc.store_scatter(L2_v, [gi], 2.0 * ln * xs)
            return 0

        lax.fori_loop(0, NG, pre3, 0)

        pltpu.sync_copy(qp_hbm.at[0], x_v)

        def zacc(i, _):
            plsc.store_scatter(acc_v, [i * 16 + iota], zf16)
            return 0

        lax.fori_loop(0, NG, zacc, 0)
        g0 = jnp.maximum(_sget(x_v, N - 1), LB)
        plsc.store_scatter(outs_v, [iota * 0], jnp.full((16,), g0, F32))

        def step(t, _):
            pltpu.sync_copy(qp_hbm.at[t - 1], ql_v)

            def mv_chunk(c, _c):
                s = c * CHUNK
                pltpu.sync_copy(src_hbm.at[pl.ds(s, CHUNK)], se_v)
                pltpu.sync_copy(dst_hbm.at[pl.ds(s, CHUNK)], de_v)

                def mv_grp(g, _g):
                    gi = g * 16 + iota
                    sv = plsc.load_gather(se_v, [gi])
                    dv = plsc.load_gather(de_v, [gi])
                    plsc.addupdate_scatter(acc_v, [dv], plsc.load_gather(x_v, [sv]))
                    return 0

                lax.fori_loop(0, CHUNK // 16, mv_grp, 0)
                return 0

            lax.fori_loop(0, E // CHUNK, mv_chunk, 0)

            def binit(i, _b):
                gi = i * 16 + iota
                disch = plsc.load_gather(x_v, [gi])
                e1 = plsc.load_gather(e1_v, [gi])
                depth = jnp.exp(e1 * _vln(disch * plsc.load_gather(A1_v, [gi])))
                depth = jnp.maximum(depth, 0.01)
                tw = plsc.load_gather(tw_v, [gi])
                bw = jnp.maximum(tw - plsc.load_gather(ss2_v, [gi]) * depth, 0.1)
                area = (tw + bw) * depth * 0.5
                wp = bw + depth * plsc.load_gather(ssb_v, [gi])
                v = plsc.load_gather(iv_v, [gi]) * jnp.exp((2.0 / 3.0) * _vln(area / wp))
                v = jnp.clip(v, 0.3, 15.0) * (5.0 / 3.0)
                invv = 1.0 / v
                kl1 = plsc.load_gather(L1_v, [gi]) * invv
                kl2 = plsc.load_gather(L2_v, [gi]) * invv
                rden = 1.0 / (kl1 + DT)
                c2 = (DT + kl2) * rden
                c3 = (kl1 - DT) * rden
                c4 = (2.0 * DT) * rden
                ql = jnp.maximum(plsc.load_gather(ql_v, [gi]), LB)
                b = c2 * plsc.load_gather(acc_v, [gi]) + c3 * disch + c4 * ql
                c1n = -((DT - kl2) * rden)
                c1n = jnp.where(gi == 0, 1.0, c1n)
                plsc.store_scatter(c1n_v, [gi], c1n)
                plsc.store_scatter(x_v, [gi], b)
                plsc.store_scatter(acc_v, [gi], zf16)
                return 0

            lax.fori_loop(0, NG, binit, 0)

            def level(l, _l):
                e0 = _sget(off_v, l)
                e1x = _sget(off_v, l + 1)
                base = e0 - lax.rem(e0, 8)
                nch = lax.div(e1x - base + (CHUNK - 1), CHUNK)

                def ch(c, _ch):
                    s = pl.multiple_of(base + c * CHUNK, 8)
                    pltpu.sync_copy(src_hbm.at[pl.ds(s, CHUNK)], se_v)
                    pltpu.sync_copy(dst_hbm.at[pl.ds(s, CHUNK)], de_v)

                    def grp(g, _g):
                        gi = g * 16 + iota
                        gidx = s + gi
                        m = jnp.logical_and(gidx >= e0, gidx < e1x)
                        sv = plsc.load_gather(se_v, [gi])
                        dv = plsc.load_gather(de_v, [gi])
                        vals = jnp.where(m, plsc.load_gather(x_v, [sv]), 0.0)
                        plsc.addupdate_scatter(acc_v, [dv], vals)
                        return 0

                    lax.fori_loop(0, CHUNK // 16, grp, 0)
                    return 0

                lax.fori_loop(0, nch, ch, 0)
                lo = l * BLOCK
                hi = jnp.minimum(lo + BLOCK, N)

                def upd(u, _u):
                    ui = lo + u * 16 + iota
                    m = ui < hi
                    bb = plsc.load_gather(x_v, [ui])
                    aa = plsc.load_gather(acc_v, [ui])
                    cc = plsc.load_gather(c1n_v, [ui])
                    plsc.store_scatter(x_v, [ui], jnp.where(m, bb - cc * aa, bb))
                    return 0

                lax.fori_loop(0, (BLOCK + 15) // 16, upd, 0)
                return 0

            lax.fori_loop(1, LEVELS, level, 0)

            def clip_body(i, _cl):
                gi = i * 16 + iota
                plsc.store_scatter(x_v, [gi],
                                   jnp.maximum(plsc.load_gather(x_v, [gi]), LB))
                plsc.store_scatter(acc_v, [gi], zf16)
                return 0

            lax.fori_loop(0, NG, clip_body, 0)
            gv = _sget(x_v, N - 1)
            plsc.store_scatter(outs_v, [jnp.full((16,), t, I32)],
                               jnp.full((16,), gv, F32))
            return 0

        lax.fori_loop(1, T, step, 0)
        pltpu.sync_copy(outs_v, out_hbm)


def kernel(q_prime, spatial_n, spatial_q, length, slope, top_width, side_slope,
           x_storage, edge_index):
    src = edge_index[0].astype(I32)
    dst = edge_index[1].astype(I32)
    lvl = dst // BLOCK
    order = jnp.argsort(lvl)
    src_s = jnp.pad(src[order], (0, EPAD - E))
    dst_s = jnp.pad(dst[order], (0, EPAD - E))
    off = jnp.searchsorted(lvl[order], jnp.arange(LEVELS + 1, dtype=I32))
    off = jnp.pad(off.astype(I32), (0, 48 - (LEVELS + 1)))

    pad1 = lambda a: jnp.pad(a.astype(F32), (0, NPAD - N))
    qp = jnp.pad(q_prime.astype(F32), ((0, 0), (0, NPAD - N)))
    out16 = _route_sc(qp, pad1(spatial_n), pad1(spatial_q), pad1(length),
                      pad1(slope), pad1(top_width), pad1(side_slope),
                      pad1(x_storage), src_s, dst_s, off)
    return out16[:T].reshape(1, T)


# speedup vs baseline: 100.6840x; 1.5873x over previous
import functools

import jax
import jax.numpy as jnp
from jax import lax
from jax.experimental import pallas as pl
from jax.experimental.pallas import tpu as pltpu
from jax.experimental.pallas import tpu_sc as plsc

N = 10000
NPAD = 10240
E = 160000
T = 9
LEVELS = 32
BLOCK = (N + LEVELS - 1) // LEVELS
DT = 3600.0
LB = 1e-4
CHUNK = 2000
EPAD = E + 2048
NG = NPAD // 16
LN2 = 0.6931471805599453
SQRT2 = 1.4142135623730951
I32 = jnp.int32
F32 = jnp.float32


def _vln(x):
    bits = lax.bitcast_convert_type(x, I32)
    e = lax.shift_right_arithmetic(bits, 23) - 127
    m = lax.bitcast_convert_type((bits & 0x7FFFFF) | 0x3F800000, F32)
    big = m > SQRT2
    m = jnp.where(big, m * 0.5, m)
    e = jnp.where(big, e + 1, e)
    z = (m - 1.0) / (m + 1.0)
    z2 = z * z
    p = 1.0 + z2 * (1.0 / 3.0 + z2 * (1.0 / 5.0 + z2 * (1.0 / 7.0 + z2 * (1.0 / 9.0))))
    return e.astype(F32) * LN2 + 2.0 * z * p


def _vsqrt(x):
    return jnp.exp(0.5 * _vln(x))


def _sget(ref, i):
    return jnp.max(plsc.load_gather(ref, [jnp.full((16,), i, I32)]))


_INTERPRET = False
_sc_mesh = plsc.VectorSubcoreMesh(core_axis_name="c", subcore_axis_name="s",
                                  num_cores=2, num_subcores=16)


@functools.partial(
    pl.kernel,
    out_type=jax.ShapeDtypeStruct((16,), F32),
    mesh=_sc_mesh,
    scratch_types=[
        pltpu.VMEM((NPAD,), F32),
        pltpu.VMEM((NPAD,), F32),
        pltpu.VMEM((NPAD,), F32),
        pltpu.VMEM((NPAD,), F32),
        pltpu.VMEM((NPAD,), F32),
        pltpu.VMEM((NPAD,), F32),
        pltpu.VMEM((NPAD,), F32),
        pltpu.VMEM((NPAD,), F32),
        pltpu.VMEM((NPAD,), F32),
        pltpu.VMEM((NPAD,), F32),
        pltpu.VMEM((NPAD,), F32),
        pltpu.VMEM((NPAD,), F32),
        pltpu.VMEM((CHUNK,), I32),
        pltpu.VMEM((CHUNK,), I32),
        pltpu.VMEM((48,), I32),
        pltpu.VMEM((16,), F32),
    ],
    compiler_params=pltpu.CompilerParams(needs_layout_passes=False),
    interpret=_INTERPRET,
)
def _route_sc(qp_hbm, n_hbm, q_hbm, len_hbm, s_hbm, tw_hbm, ss_hbm, xs_hbm,
              src_hbm, dst_hbm, off_hbm, out_hbm,
              x_v, acc_v, c1n_v, ql_v, A1_v, e1_v, iv_v, tw_v, ss2_v, ssb_v,
              L1_v, L2_v, se_v, de_v, off_v, outs_v):
    cid = lax.axis_index("c")
    sid = lax.axis_index("s")

    @pl.when(jnp.logical_and(cid == 0, sid == 0))
    def _():
        iota = lax.iota(I32, 16)
        zf16 = jnp.zeros((16,), F32)
        pltpu.sync_copy(off_hbm, off_v)

        pltpu.sync_copy(n_hbm, acc_v)
        pltpu.sync_copy(q_hbm, c1n_v)
        pltpu.sync_copy(s_hbm, ql_v)

        def pre1(i, _):
            gi = i * 16 + iota
            n_man = plsc.load_gather(acc_v, [gi]) * 0.29 + 0.01
            qsp = plsc.load_gather(c1n_v, [gi]) * 3.0
            s0 = jnp.maximum(plsc.load_gather(ql_v, [gi]), 1e-4)
            sq = _vsqrt(s0)
            plsc.store_scatter(A1_v, [gi], n_man * (qsp + 1.0) / (21.0 * sq + 1e-8))
            plsc.store_scatter(e1_v, [gi], 3.0 / (5.0 + 3.0 * qsp))
            plsc.store_scatter(iv_v, [gi], sq / n_man)
            return 0

        lax.fori_loop(0, NG, pre1, 0)
        pltpu.sync_copy(tw_hbm, tw_v)
        pltpu.sync_copy(ss_hbm, acc_v)

        def pre2(i, _):
            gi = i * 16 + iota
            ss = plsc.load_gather(acc_v, [gi])
            plsc.store_scatter(ss2_v, [gi], 2.0 * ss)
            plsc.store_scatter(ssb_v, [gi], 2.0 * _vsqrt(1.0 + ss * ss))
            return 0

        lax.fori_loop(0, NG, pre2, 0)
        pltpu.sync_copy(len_hbm, acc_v)
        pltpu.sync_copy(xs_hbm, c1n_v)

        def pre3(i, _):
            gi = i * 16 + iota
            ln = plsc.load_gather(acc_v, [gi])
            xs = plsc.load_gather(c1n_v, [gi])
            plsc.store_scatter(L1_v, [gi], 2.0 * ln * (1.0 - xs))
            plsc.store_scatter(L2_v, [gi], 2.0 * ln * xs)
            return 0

        lax.fori_loop(0, NG, pre3, 0)

        pltpu.sync_copy(qp_hbm.at[0], x_v)

        def zacc(i, _):
            plsc.store_scatter(acc_v, [i * 16 + iota], zf16)
            return 0

        lax.fori_loop(0, NG, zacc, 0)
        g0 = jnp.maximum(_sget(x_v, N - 1), LB)
        plsc.store_scatter(outs_v, [iota * 0], jnp.full((16,), g0, F32))

        def step(t, _):
            pltpu.sync_copy(qp_hbm.at[t - 1], ql_v)

            def mv_chunk(c, _c):
                s = c * CHUNK
                pltpu.sync_copy(src_hbm.at[pl.ds(s, CHUNK)], se_v)
                pltpu.sync_copy(dst_hbm.at[pl.ds(s, CHUNK)], de_v)

                @plsc.parallel_loop(0, CHUNK // 16, unroll=4)
                def _mv(g):
                    gi = g * 16 + iota
                    sv = plsc.load_gather(se_v, [gi])
                    dv = plsc.load_gather(de_v, [gi])
                    plsc.addupdate_scatter(acc_v, [dv], plsc.load_gather(x_v, [sv]))

                return 0

            lax.fori_loop(0, E // CHUNK, mv_chunk, 0)

            @plsc.parallel_loop(0, NG, unroll=2)
            def binit(i):
                gi = i * 16 + iota
                disch = plsc.load_gather(x_v, [gi])
                e1 = plsc.load_gather(e1_v, [gi])
                depth = jnp.exp(e1 * _vln(disch * plsc.load_gather(A1_v, [gi])))
                depth = jnp.maximum(depth, 0.01)
                tw = plsc.load_gather(tw_v, [gi])
                bw = jnp.maximum(tw - plsc.load_gather(ss2_v, [gi]) * depth, 0.1)
                area = (tw + bw) * depth * 0.5
                wp = bw + depth * plsc.load_gather(ssb_v, [gi])
                v = plsc.load_gather(iv_v, [gi]) * jnp.exp((2.0 / 3.0) * _vln(area / wp))
                v = jnp.clip(v, 0.3, 15.0) * (5.0 / 3.0)
                invv = 1.0 / v
                kl1 = plsc.load_gather(L1_v, [gi]) * invv
                kl2 = plsc.load_gather(L2_v, [gi]) * invv
                rden = 1.0 / (kl1 + DT)
                c2 = (DT + kl2) * rden
                c3 = (kl1 - DT) * rden
                c4 = (2.0 * DT) * rden
                ql = jnp.maximum(plsc.load_gather(ql_v, [gi]), LB)
                b = c2 * plsc.load_gather(acc_v, [gi]) + c3 * disch + c4 * ql
                c1n = -((DT - kl2) * rden)
                c1n = jnp.where(gi == 0, 1.0, c1n)
                plsc.store_scatter(c1n_v, [gi], c1n)
                plsc.store_scatter(x_v, [gi], b)
                plsc.store_scatter(acc_v, [gi], zf16)

            def level(l, _l):
                e0 = _sget(off_v, l)
                e1x = _sget(off_v, l + 1)
                base = e0 - lax.rem(e0, 8)
                nch = lax.div(e1x - base + (CHUNK - 1), CHUNK)

                def ch(c, _ch):
                    s = pl.multiple_of(base + c * CHUNK, 8)
                    pltpu.sync_copy(src_hbm.at[pl.ds(s, CHUNK)], se_v)
                    pltpu.sync_copy(dst_hbm.at[pl.ds(s, CHUNK)], de_v)

                    @plsc.parallel_loop(0, CHUNK // 16, unroll=4)
                    def _grp(g):
                        gi = g * 16 + iota
                        gidx = s + gi
                        m = jnp.logical_and(gidx >= e0, gidx < e1x)
                        sv = plsc.load_gather(se_v, [gi])
                        dv = plsc.load_gather(de_v, [gi])
                        vals = jnp.where(m, plsc.load_gather(x_v, [sv]), 0.0)
                        plsc.addupdate_scatter(acc_v, [dv], vals)

                    return 0

                lax.fori_loop(0, nch, ch, 0)
                lo = l * BLOCK
                hi = jnp.minimum(lo + BLOCK, N)

                def upd(u, _u):
                    ui = lo + u * 16 + iota
                    m = ui < hi
                    bb = plsc.load_gather(x_v, [ui])
                    aa = plsc.load_gather(acc_v, [ui])
                    cc = plsc.load_gather(c1n_v, [ui])
                    plsc.store_scatter(x_v, [ui], jnp.where(m, bb - cc * aa, bb))
                    return 0

                lax.fori_loop(0, (BLOCK + 15) // 16, upd, 0)
                return 0

            lax.fori_loop(1, LEVELS, level, 0)

            @plsc.parallel_loop(0, NG, unroll=4)
            def clip_body(i):
                gi = i * 16 + iota
                plsc.store_scatter(x_v, [gi],
                                   jnp.maximum(plsc.load_gather(x_v, [gi]), LB))
                plsc.store_scatter(acc_v, [gi], zf16)
            gv = _sget(x_v, N - 1)
            plsc.store_scatter(outs_v, [jnp.full((16,), t, I32)],
                               jnp.full((16,), gv, F32))
            return 0

        lax.fori_loop(1, T, step, 0)
        pltpu.sync_copy(outs_v, out_hbm)


def kernel(q_prime, spatial_n, spatial_q, length, slope, top_width, side_slope,
           x_storage, edge_index):
    src = edge_index[0].astype(I32)
    dst = edge_index[1].astype(I32)
    lvl = dst // BLOCK
    order = jnp.argsort(lvl)
    src_s = jnp.pad(src[order], (0, EPAD - E))
    dst_s = jnp.pad(dst[order], (0, EPAD - E))
    off = jnp.searchsorted(lvl[order], jnp.arange(LEVELS + 1, dtype=I32))
    off = jnp.pad(off.astype(I32), (0, 48 - (LEVELS + 1)))

    pad1 = lambda a: jnp.pad(a.astype(F32), (0, NPAD - N))
    qp = jnp.pad(q_prime.astype(F32), ((0, 0), (0, NPAD - N)))
    out16 = _route_sc(qp, pad1(spatial_n), pad1(spatial_q), pad1(length),
                      pad1(slope), pad1(top_width), pad1(side_slope),
                      pad1(x_storage), src_s, dst_s, off)
    return out16[:T].reshape(1, T)


# speedup vs baseline: 139.2207x; 1.3827x over previous
import functools

import jax
import jax.numpy as jnp
from jax import lax
from jax.experimental import pallas as pl
from jax.experimental.pallas import tpu as pltpu
from jax.experimental.pallas import tpu_sc as plsc

N = 10000
NPAD = 10240
E = 160000
T = 9
LEVELS = 32
BLOCK = (N + LEVELS - 1) // LEVELS
DT = 3600.0
LB = 1e-4
CHUNK = 2000
EPAD = E + 2048
NG = NPAD // 16
LN2 = 0.6931471805599453
SQRT2 = 1.4142135623730951
I32 = jnp.int32
F32 = jnp.float32


def _vln(x):
    bits = lax.bitcast_convert_type(x, I32)
    e = lax.shift_right_arithmetic(bits, 23) - 127
    m = lax.bitcast_convert_type((bits & 0x7FFFFF) | 0x3F800000, F32)
    big = m > SQRT2
    m = jnp.where(big, m * 0.5, m)
    e = jnp.where(big, e + 1, e)
    z = (m - 1.0) / (m + 1.0)
    z2 = z * z
    p = 1.0 + z2 * (1.0 / 3.0 + z2 * (1.0 / 5.0 + z2 * (1.0 / 7.0 + z2 * (1.0 / 9.0))))
    return e.astype(F32) * LN2 + 2.0 * z * p


def _vsqrt(x):
    return jnp.exp(0.5 * _vln(x))


def _sget(ref, i):
    return jnp.max(plsc.load_gather(ref, [jnp.full((16,), i, I32)]))


_INTERPRET = False
_sc_mesh = plsc.VectorSubcoreMesh(core_axis_name="c", subcore_axis_name="s",
                                  num_cores=2, num_subcores=16)


@functools.partial(
    pl.kernel,
    out_type=jax.ShapeDtypeStruct((16,), F32),
    mesh=_sc_mesh,
    scratch_types=[
        pltpu.VMEM((NPAD,), F32),
        pltpu.VMEM((NPAD,), F32),
        pltpu.VMEM((NPAD,), F32),
        pltpu.VMEM((NPAD,), F32),
        pltpu.VMEM((NPAD,), F32),
        pltpu.VMEM((NPAD,), F32),
        pltpu.VMEM((NPAD,), F32),
        pltpu.VMEM((NPAD,), F32),
        pltpu.VMEM((NPAD,), F32),
        pltpu.VMEM((NPAD,), F32),
        pltpu.VMEM((NPAD,), F32),
        pltpu.VMEM((NPAD,), F32),
        pltpu.VMEM((CHUNK,), I32),
        pltpu.VMEM((CHUNK,), I32),
        pltpu.VMEM((48,), I32),
        pltpu.VMEM((16,), F32),
        pltpu.SemaphoreType.DMA,
        pltpu.SemaphoreType.DMA,
    ],
    compiler_params=pltpu.CompilerParams(needs_layout_passes=False),
    interpret=_INTERPRET,
)
def _route_sc(qp_hbm, n_hbm, q_hbm, len_hbm, s_hbm, tw_hbm, ss_hbm, xs_hbm,
              src_hbm, dst_hbm, off_hbm, out_hbm,
              x_v, acc_v, c1n_v, ql_v, A1_v, e1_v, iv_v, tw_v, ss2_v, ssb_v,
              L1_v, L2_v, se_v, de_v, off_v, outs_v, sem_a, sem_b):
    cid = lax.axis_index("c")
    sid = lax.axis_index("s")

    @pl.when(jnp.logical_and(cid == 0, sid == 0))
    def _():
        iota = lax.iota(I32, 16)
        zf16 = jnp.zeros((16,), F32)
        pltpu.sync_copy(off_hbm, off_v)

        pltpu.sync_copy(n_hbm, acc_v)
        pltpu.sync_copy(q_hbm, c1n_v)
        pltpu.sync_copy(s_hbm, ql_v)

        def pre1(i, _):
            gi = i * 16 + iota
            n_man = plsc.load_gather(acc_v, [gi]) * 0.29 + 0.01
            qsp = plsc.load_gather(c1n_v, [gi]) * 3.0
            s0 = jnp.maximum(plsc.load_gather(ql_v, [gi]), 1e-4)
            sq = _vsqrt(s0)
            plsc.store_scatter(A1_v, [gi], n_man * (qsp + 1.0) / (21.0 * sq + 1e-8))
            plsc.store_scatter(e1_v, [gi], 3.0 / (5.0 + 3.0 * qsp))
            plsc.store_scatter(iv_v, [gi], sq / n_man)
            return 0

        lax.fori_loop(0, NG, pre1, 0)
        pltpu.sync_copy(tw_hbm, tw_v)
        pltpu.sync_copy(ss_hbm, acc_v)

        def pre2(i, _):
            gi = i * 16 + iota
            ss = plsc.load_gather(acc_v, [gi])
            plsc.store_scatter(ss2_v, [gi], 2.0 * ss)
            plsc.store_scatter(ssb_v, [gi], 2.0 * _vsqrt(1.0 + ss * ss))
            return 0

        lax.fori_loop(0, NG, pre2, 0)
        pltpu.sync_copy(len_hbm, acc_v)
        pltpu.sync_copy(xs_hbm, c1n_v)

        def pre3(i, _):
            gi = i * 16 + iota
            ln = plsc.load_gather(acc_v, [gi])
            xs = plsc.load_gather(c1n_v, [gi])
            plsc.store_scatter(L1_v, [gi], 2.0 * ln * (1.0 - xs))
            plsc.store_scatter(L2_v, [gi], 2.0 * ln * xs)
            return 0

        lax.fori_loop(0, NG, pre3, 0)

        pltpu.sync_copy(qp_hbm.at[0], x_v)

        def zacc(i, _):
            plsc.store_scatter(acc_v, [i * 16 + iota], zf16)
            return 0

        lax.fori_loop(0, NG, zacc, 0)
        g0 = jnp.maximum(_sget(x_v, N - 1), LB)
        plsc.store_scatter(outs_v, [iota * 0], jnp.full((16,), g0, F32))

        def step(t, _):
            pltpu.sync_copy(qp_hbm.at[t - 1], ql_v)

            def mv_chunk(c, _c):
                s = c * CHUNK
                ca = pltpu.async_copy(src_hbm.at[pl.ds(s, CHUNK)], se_v, sem_a)
                cb = pltpu.async_copy(dst_hbm.at[pl.ds(s, CHUNK)], de_v, sem_b)
                ca.wait()
                cb.wait()

                @plsc.parallel_loop(0, CHUNK // 16, unroll=4)
                def _mv(g):
                    gi = g * 16 + iota
                    sv = plsc.load_gather(se_v, [gi])
                    dv = plsc.load_gather(de_v, [gi])
                    plsc.addupdate_scatter(acc_v, [dv], plsc.load_gather(x_v, [sv]))

                return 0

            lax.fori_loop(0, E // CHUNK, mv_chunk, 0)

            @plsc.parallel_loop(0, NG, unroll=2)
            def binit(i):
                gi = i * 16 + iota
                disch = plsc.load_gather(x_v, [gi])
                e1 = plsc.load_gather(e1_v, [gi])
                depth = jnp.exp(e1 * _vln(disch * plsc.load_gather(A1_v, [gi])))
                depth = jnp.maximum(depth, 0.01)
                tw = plsc.load_gather(tw_v, [gi])
                bw = jnp.maximum(tw - plsc.load_gather(ss2_v, [gi]) * depth, 0.1)
                area = (tw + bw) * depth * 0.5
                wp = bw + depth * plsc.load_gather(ssb_v, [gi])
                v = plsc.load_gather(iv_v, [gi]) * jnp.exp((2.0 / 3.0) * _vln(area / wp))
                v = jnp.clip(v, 0.3, 15.0) * (5.0 / 3.0)
                invv = 1.0 / v
                kl1 = plsc.load_gather(L1_v, [gi]) * invv
                kl2 = plsc.load_gather(L2_v, [gi]) * invv
                rden = 1.0 / (kl1 + DT)
                c2 = (DT + kl2) * rden
                c3 = (kl1 - DT) * rden
                c4 = (2.0 * DT) * rden
                ql = jnp.maximum(plsc.load_gather(ql_v, [gi]), LB)
                b = c2 * plsc.load_gather(acc_v, [gi]) + c3 * disch + c4 * ql
                c1n = -((DT - kl2) * rden)
                c1n = jnp.where(gi == 0, 1.0, c1n)
                plsc.store_scatter(c1n_v, [gi], c1n)
                plsc.store_scatter(x_v, [gi], b)
                plsc.store_scatter(acc_v, [gi], zf16)

            def level(l, _l):
                e0 = _sget(off_v, l)
                e1x = _sget(off_v, l + 1)
                base = e0 - lax.rem(e0, 8)
                nch = lax.div(e1x - base + (CHUNK - 1), CHUNK)

                def ch(c, _ch):
                    s = pl.multiple_of(base + c * CHUNK, 8)
                    ca = pltpu.async_copy(src_hbm.at[pl.ds(s, CHUNK)], se_v, sem_a)
                    cb = pltpu.async_copy(dst_hbm.at[pl.ds(s, CHUNK)], de_v, sem_b)
                    ca.wait()
                    cb.wait()

                    @plsc.parallel_loop(0, CHUNK // 16, unroll=4)
                    def _grp(g):
                        gi = g * 16 + iota
                        gidx = s + gi
                        m = jnp.logical_and(gidx >= e0, gidx < e1x)
                        sv = plsc.load_gather(se_v, [gi])
                        dv = plsc.load_gather(de_v, [gi])
                        vals = jnp.where(m, plsc.load_gather(x_v, [sv]), 0.0)
                        plsc.addupdate_scatter(acc_v, [dv], vals)

                    return 0

                lax.fori_loop(0, nch, ch, 0)
                lo = l * BLOCK
                hi = jnp.minimum(lo + BLOCK, N)

                def upd(u, _u):
                    ui = lo + u * 16 + iota
                    m = ui < hi
                    bb = plsc.load_gather(x_v, [ui])
                    aa = plsc.load_gather(acc_v, [ui])
                    cc = plsc.load_gather(c1n_v, [ui])
                    plsc.store_scatter(x_v, [ui], jnp.where(m, bb - cc * aa, bb))
                    return 0

                lax.fori_loop(0, (BLOCK + 15) // 16, upd, 0)
                return 0

            lax.fori_loop(1, LEVELS, level, 0)

            @plsc.parallel_loop(0, NG, unroll=4)
            def clip_body(i):
                gi = i * 16 + iota
                plsc.store_scatter(x_v, [gi],
                                   jnp.maximum(plsc.load_gather(x_v, [gi]), LB))
                plsc.store_scatter(acc_v, [gi], zf16)
            gv = _sget(x_v, N - 1)
            plsc.store_scatter(outs_v, [jnp.full((16,), t, I32)],
                               jnp.full((16,), gv, F32))
            return 0

        lax.fori_loop(1, T, step, 0)
        pltpu.sync_copy(outs_v, out_hbm)


def kernel(q_prime, spatial_n, spatial_q, length, slope, top_width, side_slope,
           x_storage, edge_index):
    src = edge_index[0].astype(I32)
    dst = edge_index[1].astype(I32)
    lvl = dst // BLOCK
    order = jnp.argsort(lvl)
    src_s = jnp.pad(src[order], (0, EPAD - E))
    dst_s = jnp.pad(dst[order], (0, EPAD - E))
    off = jnp.searchsorted(lvl[order], jnp.arange(LEVELS + 1, dtype=I32))
    off = jnp.pad(off.astype(I32), (0, 48 - (LEVELS + 1)))

    pad1 = lambda a: jnp.pad(a.astype(F32), (0, NPAD - N))
    qp = jnp.pad(q_prime.astype(F32), ((0, 0), (0, NPAD - N)))
    out16 = _route_sc(qp, pad1(spatial_n), pad1(spatial_q), pad1(length),
                      pad1(slope), pad1(top_width), pad1(side_slope),
                      pad1(x_storage), src_s, dst_s, off)
    return out16[:T].reshape(1, T)


# speedup vs baseline: 149.3474x; 1.0727x over previous
import functools

import jax
import jax.numpy as jnp
from jax import lax
from jax.experimental import pallas as pl
from jax.experimental.pallas import tpu as pltpu
from jax.experimental.pallas import tpu_sc as plsc

N = 10000
NPAD = 10240
E = 160000
T = 9
LEVELS = 32
BLOCK = (N + LEVELS - 1) // LEVELS
DT = 3600.0
LB = 1e-4
CHUNK = 1600
EPAD = E + 2048
NG = NPAD // 16
LN2 = 0.6931471805599453
SQRT2 = 1.4142135623730951
I32 = jnp.int32
F32 = jnp.float32


def _vln(x):
    bits = lax.bitcast_convert_type(x, I32)
    e = lax.shift_right_arithmetic(bits, 23) - 127
    m = lax.bitcast_convert_type((bits & 0x7FFFFF) | 0x3F800000, F32)
    big = m > SQRT2
    m = jnp.where(big, m * 0.5, m)
    e = jnp.where(big, e + 1, e)
    z = (m - 1.0) / (m + 1.0)
    z2 = z * z
    p = 1.0 + z2 * (1.0 / 3.0 + z2 * (1.0 / 5.0 + z2 * (1.0 / 7.0 + z2 * (1.0 / 9.0))))
    return e.astype(F32) * LN2 + 2.0 * z * p


def _vsqrt(x):
    return jnp.exp(0.5 * _vln(x))


def _sget(ref, i):
    return jnp.max(plsc.load_gather(ref, [jnp.full((16,), i, I32)]))


_INTERPRET = False
_sc_mesh = plsc.VectorSubcoreMesh(core_axis_name="c", subcore_axis_name="s",
                                  num_cores=2, num_subcores=16)


@functools.partial(
    pl.kernel,
    out_type=jax.ShapeDtypeStruct((16,), F32),
    mesh=_sc_mesh,
    scratch_types=[
        pltpu.VMEM((NPAD,), F32),
        pltpu.VMEM((NPAD,), F32),
        pltpu.VMEM((NPAD,), F32),
        pltpu.VMEM((NPAD,), F32),
        pltpu.VMEM((NPAD,), F32),
        pltpu.VMEM((NPAD,), F32),
        pltpu.VMEM((NPAD,), F32),
        pltpu.VMEM((NPAD,), F32),
        pltpu.VMEM((NPAD,), F32),
        pltpu.VMEM((NPAD,), F32),
        pltpu.VMEM((NPAD,), F32),
        pltpu.VMEM((NPAD,), F32),
        pltpu.VMEM((CHUNK,), I32),
        pltpu.VMEM((CHUNK,), I32),
        pltpu.VMEM((48,), I32),
        pltpu.VMEM((16,), F32),
        pltpu.VMEM((CHUNK,), I32),
        pltpu.VMEM((CHUNK,), I32),
        pltpu.SemaphoreType.DMA,
        pltpu.SemaphoreType.DMA,
        pltpu.SemaphoreType.DMA,
        pltpu.SemaphoreType.DMA,
    ],
    compiler_params=pltpu.CompilerParams(needs_layout_passes=False),
    interpret=_INTERPRET,
)
def _route_sc(qp_hbm, n_hbm, q_hbm, len_hbm, s_hbm, tw_hbm, ss_hbm, xs_hbm,
              src_hbm, dst_hbm, off_hbm, out_hbm,
              x_v, acc_v, c1n_v, ql_v, A1_v, e1_v, iv_v, tw_v, ss2_v, ssb_v,
              L1_v, L2_v, se_v, de_v, off_v, outs_v, se2_v, de2_v,
              sem_a, sem_b, sem_c, sem_d):
    cid = lax.axis_index("c")
    sid = lax.axis_index("s")

    @pl.when(jnp.logical_and(cid == 0, sid == 0))
    def _():
        iota = lax.iota(I32, 16)
        zf16 = jnp.zeros((16,), F32)
        pltpu.sync_copy(off_hbm, off_v)

        pltpu.sync_copy(n_hbm, acc_v)
        pltpu.sync_copy(q_hbm, c1n_v)
        pltpu.sync_copy(s_hbm, ql_v)

        def pre1(i, _):
            gi = i * 16 + iota
            n_man = plsc.load_gather(acc_v, [gi]) * 0.29 + 0.01
            qsp = plsc.load_gather(c1n_v, [gi]) * 3.0
            s0 = jnp.maximum(plsc.load_gather(ql_v, [gi]), 1e-4)
            sq = _vsqrt(s0)
            plsc.store_scatter(A1_v, [gi], n_man * (qsp + 1.0) / (21.0 * sq + 1e-8))
            plsc.store_scatter(e1_v, [gi], 3.0 / (5.0 + 3.0 * qsp))
            plsc.store_scatter(iv_v, [gi], sq / n_man)
            return 0

        lax.fori_loop(0, NG, pre1, 0)
        pltpu.sync_copy(tw_hbm, tw_v)
        pltpu.sync_copy(ss_hbm, acc_v)

        def pre2(i, _):
            gi = i * 16 + iota
            ss = plsc.load_gather(acc_v, [gi])
            plsc.store_scatter(ss2_v, [gi], 2.0 * ss)
            plsc.store_scatter(ssb_v, [gi], 2.0 * _vsqrt(1.0 + ss * ss))
            return 0

        lax.fori_loop(0, NG, pre2, 0)
        pltpu.sync_copy(len_hbm, acc_v)
        pltpu.sync_copy(xs_hbm, c1n_v)

        def pre3(i, _):
            gi = i * 16 + iota
            ln = plsc.load_gather(acc_v, [gi])
            xs = plsc.load_gather(c1n_v, [gi])
            plsc.store_scatter(L1_v, [gi], 2.0 * ln * (1.0 - xs))
            plsc.store_scatter(L2_v, [gi], 2.0 * ln * xs)
            return 0

        lax.fori_loop(0, NG, pre3, 0)

        pltpu.sync_copy(qp_hbm.at[0], x_v)

        def zacc(i, _):
            plsc.store_scatter(acc_v, [i * 16 + iota], zf16)
            return 0

        lax.fori_loop(0, NG, zacc, 0)
        g0 = jnp.maximum(_sget(x_v, N - 1), LB)
        plsc.store_scatter(outs_v, [iota * 0], jnp.full((16,), g0, F32))

        def step(t, _):
            pltpu.sync_copy(qp_hbm.at[t - 1], ql_v)

            pltpu.async_copy(src_hbm.at[pl.ds(0, CHUNK)], se_v, sem_a)
            pltpu.async_copy(dst_hbm.at[pl.ds(0, CHUNK)], de_v, sem_b)
            pltpu.async_copy(src_hbm.at[pl.ds(CHUNK, CHUNK)], se2_v, sem_c)
            pltpu.async_copy(dst_hbm.at[pl.ds(CHUNK, CHUNK)], de2_v, sem_d)

            def mv_pair(cc, _c):
                c0 = 2 * cc
                sA = pl.multiple_of(jnp.minimum((c0 + 2) * CHUNK, E - CHUNK), 8)
                sB = pl.multiple_of(jnp.minimum((c0 + 3) * CHUNK, E - CHUNK), 8)
                pltpu.make_async_copy(src_hbm.at[pl.ds(0, CHUNK)], se_v, sem_a).wait()
                pltpu.make_async_copy(dst_hbm.at[pl.ds(0, CHUNK)], de_v, sem_b).wait()

                @plsc.parallel_loop(0, CHUNK // 16, unroll=4)
                def _mvA(g):
                    gi = g * 16 + iota
                    sv = plsc.load_gather(se_v, [gi])
                    dv = plsc.load_gather(de_v, [gi])
                    plsc.addupdate_scatter(acc_v, [dv], plsc.load_gather(x_v, [sv]))

                pltpu.async_copy(src_hbm.at[pl.ds(sA, CHUNK)], se_v, sem_a)
                pltpu.async_copy(dst_hbm.at[pl.ds(sA, CHUNK)], de_v, sem_b)
                pltpu.make_async_copy(src_hbm.at[pl.ds(0, CHUNK)], se2_v, sem_c).wait()
                pltpu.make_async_copy(dst_hbm.at[pl.ds(0, CHUNK)], de2_v, sem_d).wait()

                @plsc.parallel_loop(0, CHUNK // 16, unroll=4)
                def _mvB(g):
                    gi = g * 16 + iota
                    sv = plsc.load_gather(se2_v, [gi])
                    dv = plsc.load_gather(de2_v, [gi])
                    plsc.addupdate_scatter(acc_v, [dv], plsc.load_gather(x_v, [sv]))

                pltpu.async_copy(src_hbm.at[pl.ds(sB, CHUNK)], se2_v, sem_c)
                pltpu.async_copy(dst_hbm.at[pl.ds(sB, CHUNK)], de2_v, sem_d)
                return 0

            lax.fori_loop(0, E // CHUNK // 2, mv_pair, 0)
            pltpu.make_async_copy(src_hbm.at[pl.ds(0, CHUNK)], se_v, sem_a).wait()
            pltpu.make_async_copy(dst_hbm.at[pl.ds(0, CHUNK)], de_v, sem_b).wait()
            pltpu.make_async_copy(src_hbm.at[pl.ds(0, CHUNK)], se2_v, sem_c).wait()
            pltpu.make_async_copy(dst_hbm.at[pl.ds(0, CHUNK)], de2_v, sem_d).wait()

            @plsc.parallel_loop(0, NG, unroll=2)
            def binit(i):
                gi = i * 16 + iota
                disch = plsc.load_gather(x_v, [gi])
                e1 = plsc.load_gather(e1_v, [gi])
                depth = jnp.exp(e1 * _vln(disch * plsc.load_gather(A1_v, [gi])))
                depth = jnp.maximum(depth, 0.01)
                tw = plsc.load_gather(tw_v, [gi])
                bw = jnp.maximum(tw - plsc.load_gather(ss2_v, [gi]) * depth, 0.1)
                area = (tw + bw) * depth * 0.5
                wp = bw + depth * plsc.load_gather(ssb_v, [gi])
                v = plsc.load_gather(iv_v, [gi]) * jnp.exp((2.0 / 3.0) * _vln(area / wp))
                v = jnp.clip(v, 0.3, 15.0) * (5.0 / 3.0)
                invv = 1.0 / v
                kl1 = plsc.load_gather(L1_v, [gi]) * invv
                kl2 = plsc.load_gather(L2_v, [gi]) * invv
                rden = 1.0 / (kl1 + DT)
                c2 = (DT + kl2) * rden
                c3 = (kl1 - DT) * rden
                c4 = (2.0 * DT) * rden
                ql = jnp.maximum(plsc.load_gather(ql_v, [gi]), LB)
                b = c2 * plsc.load_gather(acc_v, [gi]) + c3 * disch + c4 * ql
                c1n = -((DT - kl2) * rden)
                c1n = jnp.where(gi == 0, 1.0, c1n)
                plsc.store_scatter(c1n_v, [gi], c1n)
                plsc.store_scatter(x_v, [gi], b)
                plsc.store_scatter(acc_v, [gi], zf16)

            def level(l, _l):
                e0 = _sget(off_v, l)
                e1x = _sget(off_v, l + 1)
                base = e0 - lax.rem(e0, 8)
                nch = lax.div(e1x - base + (CHUNK - 1), CHUNK)

                def ch(c, _ch):
                    s = pl.multiple_of(base + c * CHUNK, 8)
                    ca = pltpu.async_copy(src_hbm.at[pl.ds(s, CHUNK)], se_v, sem_a)
                    cb = pltpu.async_copy(dst_hbm.at[pl.ds(s, CHUNK)], de_v, sem_b)
                    ca.wait()
                    cb.wait()

                    @plsc.parallel_loop(0, CHUNK // 16, unroll=4)
                    def _grp(g):
                        gi = g * 16 + iota
                        gidx = s + gi
                        m = jnp.logical_and(gidx >= e0, gidx < e1x)
                        sv = plsc.load_gather(se_v, [gi])
                        dv = plsc.load_gather(de_v, [gi])
                        vals = jnp.where(m, plsc.load_gather(x_v, [sv]), 0.0)
                        plsc.addupdate_scatter(acc_v, [dv], vals)

                    return 0

                lax.fori_loop(0, nch, ch, 0)
                lo = l * BLOCK
                hi = jnp.minimum(lo + BLOCK, N)

                def upd(u, _u):
                    ui = lo + u * 16 + iota
                    m = ui < hi
                    bb = plsc.load_gather(x_v, [ui])
                    aa = plsc.load_gather(acc_v, [ui])
                    cc = plsc.load_gather(c1n_v, [ui])
                    plsc.store_scatter(x_v, [ui], jnp.where(m, bb - cc * aa, bb))
                    return 0

                lax.fori_loop(0, (BLOCK + 15) // 16, upd, 0)
                return 0

            lax.fori_loop(1, LEVELS, level, 0)

            @plsc.parallel_loop(0, NG, unroll=4)
            def clip_body(i):
                gi = i * 16 + iota
                plsc.store_scatter(x_v, [gi],
                                   jnp.maximum(plsc.load_gather(x_v, [gi]), LB))
                plsc.store_scatter(acc_v, [gi], zf16)
            gv = _sget(x_v, N - 1)
            plsc.store_scatter(outs_v, [jnp.full((16,), t, I32)],
                               jnp.full((16,), gv, F32))
            return 0

        lax.fori_loop(1, T, step, 0)
        pltpu.sync_copy(outs_v, out_hbm)


def kernel(q_prime, spatial_n, spatial_q, length, slope, top_width, side_slope,
           x_storage, edge_index):
    src = edge_index[0].astype(I32)
    dst = edge_index[1].astype(I32)
    lvl = dst // BLOCK
    order = jnp.argsort(lvl)
    src_s = jnp.pad(src[order], (0, EPAD - E))
    dst_s = jnp.pad(dst[order], (0, EPAD - E))
    off = jnp.searchsorted(lvl[order], jnp.arange(LEVELS + 1, dtype=I32))
    off = jnp.pad(off.astype(I32), (0, 48 - (LEVELS + 1)))

    pad1 = lambda a: jnp.pad(a.astype(F32), (0, NPAD - N))
    qp = jnp.pad(q_prime.astype(F32), ((0, 0), (0, NPAD - N)))
    out16 = _route_sc(qp, pad1(spatial_n), pad1(spatial_q), pad1(length),
                      pad1(slope), pad1(top_width), pad1(side_slope),
                      pad1(x_storage), src_s, dst_s, off)
    return out16[:T].reshape(1, T)
